# baseline (speedup 1.0000x reference)
import functools

import jax
import jax.numpy as jnp
from jax import lax
from jax.experimental import pallas as pl
from jax.experimental.pallas import tpu as pltpu
from jax.experimental.pallas import tpu_sc as plsc

_MAXD = 128
_H = 16
_Q = 2048
_K = 2048
_EXT = 4096
_BANDP = 272
_BAND_LO = _Q - 1 - _MAXD
_NCLS = 16
_ROWLEN = 3968


def _build_body(wt_hbm, eflat_hbm, band_v, e_v, sem):
    wid = lax.axis_index("s") * 2 + lax.axis_index("c")
    h = wid >> 1
    p = wid & 1

    pltpu.sync_copy(
        wt_hbm.at[pl.ds(pl.multiple_of(h * _BANDP, 8), _BANDP)], band_v
    )

    zeros = jnp.zeros((16,), jnp.float32)
    v_lo = zeros + band_v[pl.ds(0, 16)][0]
    v_hi = zeros + band_v[pl.ds(2 * _MAXD, 16)][0]

    for r in range(8):
        roff = r * _EXT

        def fill_lo(c, carry, roff=roff):
            e_v[pl.ds(roff + c * 16, 16)] = v_lo
            return carry

        def fill_hi(c, carry, roff=roff):
            e_v[pl.ds(roff + 2160 + c * 16, 16)] = v_hi
            return carry

        lax.fori_loop(0, 1920 // 16, fill_lo, 0)
        lax.fori_loop(0, (_EXT - 2160) // 16, fill_hi, 0)
        for c in range(_BANDP // 16):
            e_v[pl.ds(roff + _BAND_LO - r + c * 16, 16)] = band_v[
                pl.ds(c * 16, 16)
            ]

    def emit(i, carry):
        ci = i >> 3
        j = i & 7
        cls = 2 * ci + p
        s0 = (_MAXD - 1) - 8 * cls - j
        r = s0 & 7
        src = pl.multiple_of(r * _EXT + (s0 - r), 8)
        dst = pl.multiple_of(((h * _NCLS + cls) * 8 + j) * _ROWLEN, 8)
        pltpu.async_copy(
            e_v.at[pl.ds(src, _ROWLEN)], eflat_hbm.at[pl.ds(dst, _ROWLEN)], sem
        )

        @pl.when(i >= 4)
        def _wait_one():
            pltpu.make_async_copy(
                e_v.at[pl.ds(0, _ROWLEN)], eflat_hbm.at[pl.ds(0, _ROWLEN)], sem
            ).wait()

        return carry

    lax.fori_loop(0, 64, emit, 0)
    for _ in range(4):
        pltpu.make_async_copy(
            e_v.at[pl.ds(0, _ROWLEN)], eflat_hbm.at[pl.ds(0, _ROWLEN)], sem
        ).wait()


def _emit_body(e4_hbm, out_hbm, sem):
    wid = lax.axis_index("s") * 2 + lax.axis_index("c")
    h = wid >> 1
    p = wid & 1

    def emit(i, carry):
        ci = i >> 4
        t = i & 15
        cls = 2 * ci + p
        colbase = pl.multiple_of(128 * (15 - t), 128)
        pltpu.async_copy(
            e4_hbm.at[h, cls].at[:, pl.ds(colbase, _K)],
            out_hbm.at[h, cls + 16 * t],
            sem,
        )

        @pl.when(i >= 8)
        def _wait_one():
            pltpu.make_async_copy(
                e4_hbm.at[0, 0].at[:, pl.ds(0, _K)], out_hbm.at[0, 0], sem
            ).wait()

        return carry

    lax.fori_loop(0, 128, emit, 0)
    for _ in range(8):
        pltpu.make_async_copy(
            e4_hbm.at[0, 0].at[:, pl.ds(0, _K)], out_hbm.at[0, 0], sem
        ).wait()


def kernel(query_length, key_length, W):
    wt = jnp.concatenate(
        [W.T, jnp.broadcast_to(W.T[:, -1:], (_H, _BANDP - (2 * _MAXD + 1)))],
        axis=1,
    ).reshape(-1)
    mesh = plsc.VectorSubcoreMesh(core_axis_name="c", subcore_axis_name="s")

    build = functools.partial(
        pl.kernel,
        mesh=mesh,
        out_type=jax.ShapeDtypeStruct((_H * _NCLS * 8 * _ROWLEN,), jnp.float32),
        scratch_types=[
            pltpu.VMEM((_BANDP,), jnp.float32),
            pltpu.VMEM((8 * _EXT,), jnp.float32),
            pltpu.SemaphoreType.DMA,
        ],
    )(_build_body)
    e4 = build(wt).reshape(_H, _NCLS, 8, _ROWLEN)

    emit = functools.partial(
        pl.kernel,
        mesh=mesh,
        out_type=jax.ShapeDtypeStruct((_H, _Q // 8, 8, _K), jnp.float32),
        scratch_types=[
            pltpu.SemaphoreType.DMA,
        ],
    )(_emit_body)
    out = emit(e4)
    return out.reshape(1, _H, _Q, _K)


# speedup vs baseline: 47.4480x; 47.4480x over previous
import functools

import jax
import jax.numpy as jnp
from jax import lax
from jax.experimental import pallas as pl
from jax.experimental.pallas import tpu as pltpu
from jax.experimental.pallas import tpu_sc as plsc

_MAXD = 128
_H = 16
_Q = 2048
_K = 2048
_EXT = 4096
_BANDP = 272
_BAND_LO = _Q - 1 - _MAXD
_NCLS = 16
_ROWLEN = 3968


def _build_body(wt_hbm, eflat_hbm, band_v, e_v, sem):
    wid = lax.axis_index("s") * 2 + lax.axis_index("c")
    h = wid >> 1
    p = wid & 1

    pltpu.sync_copy(
        wt_hbm.at[pl.ds(pl.multiple_of(h * _BANDP, 8), _BANDP)], band_v
    )

    zeros = jnp.zeros((16,), jnp.float32)
    v_lo = zeros + band_v[pl.ds(0, 16)][0]
    v_hi = zeros + band_v[pl.ds(2 * _MAXD, 16)][0]

    for r in range(8):
        roff = r * _EXT

        def fill_lo(c, carry, roff=roff):
            e_v[pl.ds(roff + c * 16, 16)] = v_lo
            return carry

        def fill_hi(c, carry, roff=roff):
            e_v[pl.ds(roff + 2160 + c * 16, 16)] = v_hi
            return carry

        lax.fori_loop(0, 1920 // 16, fill_lo, 0)
        lax.fori_loop(0, (_EXT - 2160) // 16, fill_hi, 0)
        for c in range(_BANDP // 16):
            e_v[pl.ds(roff + _BAND_LO - r + c * 16, 16)] = band_v[
                pl.ds(c * 16, 16)
            ]

    def emit(i, carry):
        ci = i >> 3
        j = i & 7
        cls = 2 * ci + p
        s0 = (_MAXD - 1) - 8 * cls - j
        r = s0 & 7
        src = pl.multiple_of(r * _EXT + (s0 - r), 8)
        dst = pl.multiple_of(((h * _NCLS + cls) * 8 + j) * _ROWLEN, 8)
        pltpu.async_copy(
            e_v.at[pl.ds(src, _ROWLEN)], eflat_hbm.at[pl.ds(dst, _ROWLEN)], sem
        )

        @pl.when(i >= 4)
        def _wait_one():
            pltpu.make_async_copy(
                e_v.at[pl.ds(0, _ROWLEN)], eflat_hbm.at[pl.ds(0, _ROWLEN)], sem
            ).wait()

        return carry

    lax.fori_loop(0, 64, emit, 0)
    for _ in range(4):
        pltpu.make_async_copy(
            e_v.at[pl.ds(0, _ROWLEN)], eflat_hbm.at[pl.ds(0, _ROWLEN)], sem
        ).wait()


def _emit_body(e4_hbm, out_hbm, ev0, ev1, ev2, ev3, sem_in, sem_out):
    wid = lax.axis_index("s") * 2 + lax.axis_index("c")
    h = wid >> 1
    p = wid & 1
    evs = [ev0, ev1, ev2, ev3]

    for ci in range(4):
        pltpu.async_copy(e4_hbm.at[h, 2 * ci + p], evs[ci], sem_in)

    for ci in range(8):
        ev = evs[ci % 4]
        cls = 2 * ci + p
        pltpu.make_async_copy(e4_hbm.at[0, 0], ev, sem_in).wait()

        def fire_t(t, carry, ev=ev, cls=cls):
            colbase = pl.multiple_of(128 * (15 - t), 128)
            pltpu.async_copy(
                ev.at[:, pl.ds(colbase, _K)], out_hbm.at[h, cls + 16 * t],
                sem_out,
            )
            return carry

        lax.fori_loop(0, 16, fire_t, 0)

        if ci >= 3 and ci + 1 < 8:
            for _ in range(16):
                pltpu.make_async_copy(
                    ev0.at[:, pl.ds(0, _K)], out_hbm.at[0, 0], sem_out
                ).wait()
            pltpu.async_copy(e4_hbm.at[h, 2 * (ci + 1) + p], evs[(ci + 1) % 4], sem_in)

    for _ in range(4 * 16):
        pltpu.make_async_copy(
            ev0.at[:, pl.ds(0, _K)], out_hbm.at[0, 0], sem_out
        ).wait()


def kernel(query_length, key_length, W):
    wt = jnp.concatenate(
        [W.T, jnp.broadcast_to(W.T[:, -1:], (_H, _BANDP - (2 * _MAXD + 1)))],
        axis=1,
    ).reshape(-1)
    mesh = plsc.VectorSubcoreMesh(core_axis_name="c", subcore_axis_name="s")

    build = functools.partial(
        pl.kernel,
        mesh=mesh,
        out_type=jax.ShapeDtypeStruct((_H * _NCLS * 8 * _ROWLEN,), jnp.float32),
        scratch_types=[
            pltpu.VMEM((_BANDP,), jnp.float32),
            pltpu.VMEM((8 * _EXT,), jnp.float32),
            pltpu.SemaphoreType.DMA,
        ],
    )(_build_body)
    e4 = build(wt).reshape(_H, _NCLS, 8, _ROWLEN)

    emit = functools.partial(
        pl.kernel,
        mesh=mesh,
        out_type=jax.ShapeDtypeStruct((_H, _Q // 8, 8, _K), jnp.float32),
        scratch_types=[
            pltpu.VMEM((8, _ROWLEN), jnp.float32),
            pltpu.VMEM((8, _ROWLEN), jnp.float32),
            pltpu.VMEM((8, _ROWLEN), jnp.float32),
            pltpu.VMEM((8, _ROWLEN), jnp.float32),
            pltpu.SemaphoreType.DMA,
            pltpu.SemaphoreType.DMA,
        ],
    )(_emit_body)
    out = emit(e4)
    return out.reshape(1, _H, _Q, _K)


# speedup vs baseline: 66.6114x; 1.4039x over previous
import functools

import jax
import jax.numpy as jnp
from jax import lax
from jax.experimental import pallas as pl
from jax.experimental.pallas import tpu as pltpu
from jax.experimental.pallas import tpu_sc as plsc

_MAXD = 128
_H = 16
_Q = 2048
_K = 2048
_EXT = 4096
_BANDP = 272
_BAND_LO = _Q - 1 - _MAXD
_NCLS = 16
_SLEN = 896
_SBASE = 1663


def _build_body(wt_hbm, eflat_hbm, band_v, e_v, sem):
    wid = lax.axis_index("s") * 2 + lax.axis_index("c")
    h = wid >> 1
    p = wid & 1

    pltpu.sync_copy(
        wt_hbm.at[pl.ds(pl.multiple_of(h * _BANDP, 8), _BANDP)], band_v
    )

    zeros = jnp.zeros((16,), jnp.float32)
    v_lo = zeros + band_v[pl.ds(0, 16)][0]
    v_hi = zeros + band_v[pl.ds(2 * _MAXD, 16)][0]

    for r in range(8):
        roff = r * _EXT

        def fill_lo(c, carry, roff=roff):
            e_v[pl.ds(roff + 1520 + c * 16, 16)] = v_lo
            return carry

        def fill_hi(c, carry, roff=roff):
            e_v[pl.ds(roff + 2176 + c * 16, 16)] = v_hi
            return carry

        lax.fori_loop(0, (1920 - 1520) // 16, fill_lo, 0)
        lax.fori_loop(0, (2560 - 2176) // 16, fill_hi, 0)
        for c in range(_BANDP // 16):
            e_v[pl.ds(roff + _BAND_LO - r + c * 16, 16)] = band_v[
                pl.ds(c * 16, 16)
            ]

    def emit(i, carry):
        ci = i >> 3
        j = i & 7
        cls = 2 * ci + p
        s0 = _SBASE - 8 * cls - j
        r = s0 & 7
        src = pl.multiple_of(r * _EXT + (s0 - r), 8)
        dst = pl.multiple_of(((h * _NCLS + cls) * 8 + j) * _SLEN, 8)
        pltpu.async_copy(
            e_v.at[pl.ds(src, _SLEN)], eflat_hbm.at[pl.ds(dst, _SLEN)], sem
        )

        @pl.when(i >= 4)
        def _wait_one():
            pltpu.make_async_copy(
                e_v.at[pl.ds(0, _SLEN)], eflat_hbm.at[pl.ds(0, _SLEN)], sem
            ).wait()

        return carry

    lax.fori_loop(0, 64, emit, 0)
    for _ in range(4):
        pltpu.make_async_copy(
            e_v.at[pl.ds(0, _SLEN)], eflat_hbm.at[pl.ds(0, _SLEN)], sem
        ).wait()


def _emit_body(s4_hbm, out_hbm, sv0, sv1, sv2, sv3, sem_in, sem_out):
    wid = lax.axis_index("s") * 2 + lax.axis_index("c")
    h = wid >> 1
    p = wid & 1
    svs = [sv0, sv1, sv2, sv3]

    def drain_outs(n, svs0=None):
        ref = svs0 if svs0 is not None else sv0

        def wait_one(i, carry):
            pltpu.make_async_copy(
                ref.at[:, pl.ds(0, 512)],
                out_hbm.at[0, 0].at[:, pl.ds(0, 512)],
                sem_out,
            ).wait()
            return carry

        lax.fori_loop(0, n, wait_one, 0)

    for ci in range(4):
        pltpu.async_copy(s4_hbm.at[h, 2 * ci + p], svs[ci], sem_in)

    for ci in range(8):
        sv = svs[ci % 4]
        cls = 2 * ci + p
        pltpu.make_async_copy(s4_hbm.at[0, 0], sv, sem_in).wait()

        def fire_t(t, carry, sv=sv, cls=cls):
            m = cls + 16 * t
            kw = jnp.clip(128 * (t - 1), 0, 1536)
            nlo = kw >> 7
            srcoff = pl.multiple_of(kw + 384 - 128 * t, 128)
            pltpu.async_copy(
                sv.at[:, pl.ds(srcoff, 512)],
                out_hbm.at[h, m].at[:, pl.ds(pl.multiple_of(kw, 128), 512)],
                sem_out,
            )

            def fire_const(d, carry2):
                islo = d < nlo
                dtile = jnp.where(islo, d, d + 4) * 128
                srcc = jnp.where(islo, 0, _SLEN - 128)
                pltpu.async_copy(
                    sv.at[:, pl.ds(pl.multiple_of(srcc, 128), 128)],
                    out_hbm.at[h, m].at[
                        :, pl.ds(pl.multiple_of(dtile, 128), 128)
                    ],
                    sem_out,
                )
                return carry2

            lax.fori_loop(0, 12, fire_const, 0)
            return carry

        lax.fori_loop(0, 16, fire_t, 0)

        if ci >= 3 and ci + 1 < 8:
            drain_outs(64)
            pltpu.async_copy(
                s4_hbm.at[h, 2 * (ci + 1) + p], svs[(ci + 1) % 4], sem_in
            )

    drain_outs(4 * 64)


def kernel(query_length, key_length, W):
    wt = jnp.concatenate(
        [W.T, jnp.broadcast_to(W.T[:, -1:], (_H, _BANDP - (2 * _MAXD + 1)))],
        axis=1,
    ).reshape(-1)
    mesh = plsc.VectorSubcoreMesh(core_axis_name="c", subcore_axis_name="s")

    build = functools.partial(
        pl.kernel,
        mesh=mesh,
        out_type=jax.ShapeDtypeStruct((_H * _NCLS * 8 * _SLEN,), jnp.float32),
        scratch_types=[
            pltpu.VMEM((_BANDP,), jnp.float32),
            pltpu.VMEM((8 * _EXT,), jnp.float32),
            pltpu.SemaphoreType.DMA,
        ],
    )(_build_body)
    s4 = build(wt).reshape(_H, _NCLS, 8, _SLEN)

    emit = functools.partial(
        pl.kernel,
        mesh=mesh,
        out_type=jax.ShapeDtypeStruct((_H, _Q // 8, 8, _K), jnp.float32),
        scratch_types=[
            pltpu.VMEM((8, _SLEN), jnp.float32),
            pltpu.VMEM((8, _SLEN), jnp.float32),
            pltpu.VMEM((8, _SLEN), jnp.float32),
            pltpu.VMEM((8, _SLEN), jnp.float32),
            pltpu.SemaphoreType.DMA,
            pltpu.SemaphoreType.DMA,
        ],
    )(_emit_body)
    out = emit(s4)
    return out.reshape(1, _H, _Q, _K)
